# baseline (speedup 1.0000x reference)
import jax
import jax.numpy as jnp
from jax import lax
from jax.experimental import pallas as pl
from jax.experimental.pallas import tpu as pltpu

_ITEM_NUM = 1000000
_LIST_LEN = 20
_N_DMA = 8
_ROWS = _ITEM_NUM // _N_DMA


def kernel(x, item_list):
    def body(in_hbm, out_hbm, *sems):
        handles = []
        for k in range(_N_DMA):
            handles.append(pltpu.make_async_copy(
                in_hbm.at[pl.ds(2 + k * _ROWS, _ROWS), :],
                out_hbm.at[pl.ds(k * _ROWS, _ROWS), :],
                sems[k]))
        for h in handles:
            h.start()
        for h in handles:
            h.wait()

    return pl.pallas_call(
        body,
        in_specs=[pl.BlockSpec(memory_space=pl.ANY)],
        out_specs=pl.BlockSpec(memory_space=pl.ANY),
        out_shape=jax.ShapeDtypeStruct((_ITEM_NUM, _LIST_LEN), jnp.float32),
        scratch_shapes=[pltpu.SemaphoreType.DMA] * _N_DMA,
    )(item_list)


# speedup vs baseline: 18.3778x; 18.3778x over previous
import jax
import jax.numpy as jnp
from jax.experimental import pallas as pl
from jax.experimental.pallas import tpu as pltpu

_ITEM_NUM = 1000000
_LIST_LEN = 20
_R = 8000
_G = _ITEM_NUM // _R


def kernel(x, item_list):
    def body(a_ref, b_ref, o_ref):
        o_ref[0:_R - 2, :] = a_ref[2:_R, :]
        o_ref[_R - 2:_R, :] = b_ref[0:2, :]

    return pl.pallas_call(
        body,
        grid=(_G,),
        in_specs=[
            pl.BlockSpec((_R, _LIST_LEN), lambda i: (i, 0)),
            pl.BlockSpec((8, _LIST_LEN), lambda i: ((_R // 8) * (i + 1), 0)),
        ],
        out_specs=pl.BlockSpec((_R, _LIST_LEN), lambda i: (i, 0)),
        out_shape=jax.ShapeDtypeStruct((_ITEM_NUM, _LIST_LEN), jnp.float32),
    )(item_list, item_list)


# speedup vs baseline: 18.4359x; 1.0032x over previous
import jax
import jax.numpy as jnp
from jax.experimental import pallas as pl
from jax.experimental.pallas import tpu as pltpu

_ITEM_NUM = 1000000
_LIST_LEN = 20
_R = 20000
_G = _ITEM_NUM // _R


def kernel(x, item_list):
    def body(a_ref, b_ref, o_ref):
        o_ref[0:_R - 2, :] = a_ref[2:_R, :]
        o_ref[_R - 2:_R, :] = b_ref[0:2, :]

    return pl.pallas_call(
        body,
        grid=(_G,),
        in_specs=[
            pl.BlockSpec((_R, _LIST_LEN), lambda i: (i, 0)),
            pl.BlockSpec((8, _LIST_LEN), lambda i: ((_R // 8) * (i + 1), 0)),
        ],
        out_specs=pl.BlockSpec((_R, _LIST_LEN), lambda i: (i, 0)),
        out_shape=jax.ShapeDtypeStruct((_ITEM_NUM, _LIST_LEN), jnp.float32),
    )(item_list, item_list)
